# Initial kernel scaffold; baseline (speedup 1.0000x reference)
import jax
import jax.numpy as jnp
from jax.experimental import pallas as pl


def kernel(x, Wr, br, W1, b1, W2, b2):
    raise NotImplementedError("write your pallas kernel here")


# speedup vs baseline: 1.0756x; 1.0756x over previous
import functools

import jax
import jax.numpy as jnp
from jax import lax
from jax.experimental import pallas as pl
from jax.experimental.pallas import tpu as pltpu
from jax.experimental.pallas import tpu_sc as plsc

_N = 4096
_C = 1024
_E = 8
_HID = 4096
_CAP = 640
_TBLK = 512
_NBLK = _N // _TBLK
_KBLK = 512
_KN = _HID // _KBLK
_NC, _NS = 2, 16
_NW = _NC * _NS
_TOK_W = _N // _NW
_ROWS = _E * _CAP + _NW


def _router_body(x_ref, wr_ref, br_ref, dst_ref, g_ref, scale_ref, aux_ref,
                 counts_ref, imp_ref):
    i = pl.program_id(0)

    @pl.when(i == 0)
    def _():
        counts_ref[...] = jnp.zeros_like(counts_ref)
        imp_ref[...] = jnp.zeros_like(imp_ref)

    xb = x_ref[...]
    logits = jnp.dot(xb, wr_ref[...], preferred_element_type=jnp.float32)
    logits = logits + br_ref[...]
    m = jnp.max(logits, axis=-1, keepdims=True)
    ex = jnp.exp(logits - m)
    probs = ex / jnp.sum(ex, axis=-1, keepdims=True)

    is_max = (logits == m).astype(jnp.float32)
    le = (lax.broadcasted_iota(jnp.int32, (_E, _E), 0)
          <= lax.broadcasted_iota(jnp.int32, (_E, _E), 1)).astype(jnp.float32)
    nmax_incl = jnp.dot(is_max, le, preferred_element_type=jnp.float32)
    first = is_max * (nmax_incl == 1.0).astype(jnp.float32)

    eidx_col = lax.broadcasted_iota(jnp.int32, (_E, 1), 0).astype(jnp.float32)
    top_idx = jnp.dot(first, eidx_col, preferred_element_type=jnp.float32)
    top_val = jnp.sum(probs * first, axis=-1, keepdims=True)

    tril = (lax.broadcasted_iota(jnp.int32, (_TBLK, _TBLK), 0)
            >= lax.broadcasted_iota(jnp.int32, (_TBLK, _TBLK), 1)
            ).astype(jnp.float32)
    incl = jnp.dot(tril, first, preferred_element_type=jnp.float32)
    incl = incl + counts_ref[...]
    pos = jnp.sum(incl * first, axis=-1, keepdims=True) - 1.0

    counts_ref[...] += jnp.sum(first, axis=0, keepdims=True)
    imp_ref[...] += jnp.sum(probs, axis=0, keepdims=True)

    keep = pos < float(_CAP)
    posi = pos.astype(jnp.int32)
    eidx = top_idx.astype(jnp.int32)
    tok = i * _TBLK + lax.broadcasted_iota(jnp.int32, (_TBLK, 1), 0)
    garbage = _E * _CAP + tok // _TOK_W
    dst_ref[...] = jnp.where(keep, eidx * _CAP + posi, garbage)
    g_ref[...] = eidx * _CAP + jnp.minimum(posi, _CAP - 1)
    scale_ref[...] = jnp.where(keep, top_val, 0.0)

    @pl.when(i == _NBLK - 1)
    def _():
        load = jnp.minimum(counts_ref[...], float(_CAP)) / float(_N)
        imp = imp_ref[...] / float(_N)
        aux_ref[...] = jnp.sum(imp * load, keepdims=True) * float(_E)


def _router(xf, Wr, br2):
    return pl.pallas_call(
        _router_body,
        grid=(_NBLK,),
        in_specs=[
            pl.BlockSpec((_TBLK, _C), lambda i: (i, 0)),
            pl.BlockSpec((_C, _E), lambda i: (0, 0)),
            pl.BlockSpec((1, _E), lambda i: (0, 0)),
        ],
        out_specs=[
            pl.BlockSpec((_TBLK, 1), lambda i: (i, 0)),
            pl.BlockSpec((_TBLK, 1), lambda i: (i, 0)),
            pl.BlockSpec((_TBLK, 1), lambda i: (i, 0)),
            pl.BlockSpec((1, 1), lambda i: (0, 0)),
        ],
        out_shape=[
            jax.ShapeDtypeStruct((_N, 1), jnp.int32),
            jax.ShapeDtypeStruct((_N, 1), jnp.int32),
            jax.ShapeDtypeStruct((_N, 1), jnp.float32),
            jax.ShapeDtypeStruct((1, 1), jnp.float32),
        ],
        scratch_shapes=[
            pltpu.VMEM((1, _E), jnp.float32),
            pltpu.VMEM((1, _E), jnp.float32),
        ],
        compiler_params=pltpu.CompilerParams(
            dimension_semantics=("arbitrary",)),
    )(xf, Wr, br2)


_CHUNK = 64


@functools.lru_cache(maxsize=None)
def _build_sc_kernels():
    mesh = plsc.VectorSubcoreMesh(core_axis_name="c", subcore_axis_name="s")
    nrow = _TOK_W // _CHUNK
    scratch = [
        pltpu.VMEM((nrow, _CHUNK), jnp.int32),
        pltpu.VMEM((_CHUNK, _C), jnp.float32),
        pltpu.SemaphoreType.DMA,
    ]

    @functools.partial(
        pl.kernel, mesh=mesh,
        out_type=jax.ShapeDtypeStruct((_ROWS, _C), jnp.float32),
        scratch_types=scratch,
    )
    def sc_scatter(x_hbm, dst_hbm, buf_hbm, idx_v, rows_v, sem):
        wid = lax.axis_index("s") * _NC + lax.axis_index("c")
        pltpu.sync_copy(dst_hbm.at[pl.ds(wid * nrow, nrow)], idx_v)
        for j in range(nrow):
            base = wid * _TOK_W + j * _CHUNK
            pltpu.sync_copy(x_hbm.at[pl.ds(base, _CHUNK)], rows_v)
            pltpu.async_copy(rows_v, buf_hbm.at[idx_v.at[j]], sem).wait()

    @functools.partial(
        pl.kernel, mesh=mesh,
        out_type=jax.ShapeDtypeStruct((_N, _C), jnp.float32),
        scratch_types=scratch,
    )
    def sc_gather(rows_hbm, g_hbm, out_hbm, idx_v, rows_v, sem):
        wid = lax.axis_index("s") * _NC + lax.axis_index("c")
        pltpu.sync_copy(g_hbm.at[pl.ds(wid * nrow, nrow)], idx_v)
        for j in range(nrow):
            base = wid * _TOK_W + j * _CHUNK
            pltpu.async_copy(rows_hbm.at[idx_v.at[j]], rows_v, sem).wait()
            pltpu.sync_copy(rows_v, out_hbm.at[pl.ds(base, _CHUNK)])

    return sc_scatter, sc_gather


def _ffn_body(buf_ref, w1_ref, b1_ref, w2_ref, b2_ref, out_ref):
    k = pl.program_id(1)
    h = jnp.dot(buf_ref[...], w1_ref[0], preferred_element_type=jnp.float32)
    h = jnp.maximum(h + b1_ref[0], 0.0)
    contrib = jnp.dot(h, w2_ref[0], preferred_element_type=jnp.float32)

    @pl.when(k == 0)
    def _():
        out_ref[...] = contrib + b2_ref[0]

    @pl.when(k > 0)
    def _():
        out_ref[...] += contrib


def _ffn(buf, W1, b1, W2, b2):
    return pl.pallas_call(
        _ffn_body,
        grid=(_E, _KN),
        in_specs=[
            pl.BlockSpec((_CAP, _C), lambda e, k: (e, 0)),
            pl.BlockSpec((1, _C, _KBLK), lambda e, k: (e, 0, k)),
            pl.BlockSpec((1, 1, _KBLK), lambda e, k: (e, 0, k)),
            pl.BlockSpec((1, _KBLK, _C), lambda e, k: (e, k, 0)),
            pl.BlockSpec((1, 1, _C), lambda e, k: (e, 0, 0)),
        ],
        out_specs=pl.BlockSpec((_CAP, _C), lambda e, k: (e, 0)),
        out_shape=jax.ShapeDtypeStruct((_E * _CAP, _C), jnp.float32),
        compiler_params=pltpu.CompilerParams(
            dimension_semantics=("parallel", "arbitrary")),
    )(buf, W1, b1.reshape(_E, 1, _HID), W2, b2.reshape(_E, 1, _C))


def _scale_body(x_ref, s_ref, o_ref):
    o_ref[...] = x_ref[...] * s_ref[...]


def _scale_mul(gathered, scale):
    return pl.pallas_call(
        _scale_body,
        grid=(_NBLK,),
        in_specs=[
            pl.BlockSpec((_TBLK, _C), lambda i: (i, 0)),
            pl.BlockSpec((_TBLK, 1), lambda i: (i, 0)),
        ],
        out_specs=pl.BlockSpec((_TBLK, _C), lambda i: (i, 0)),
        out_shape=jax.ShapeDtypeStruct((_N, _C), jnp.float32),
        compiler_params=pltpu.CompilerParams(
            dimension_semantics=("parallel",)),
    )(gathered, scale)


def kernel(x, Wr, br, W1, b1, W2, b2):
    B, T, C = x.shape
    xf = x.reshape(_N, C)
    dst, g, scale, aux = _router(xf, Wr, br.reshape(1, _E))
    dst_w = dst.reshape(_NW * (_TOK_W // _CHUNK), _CHUNK)
    g_w = g.reshape(_NW * (_TOK_W // _CHUNK), _CHUNK)
    sc_scatter, sc_gather = _build_sc_kernels()
    buf = sc_scatter(xf, dst_w)
    out_rows = _ffn(buf, W1, b1, W2, b2)
    gathered = sc_gather(out_rows, g_w)
    out = _scale_mul(gathered, scale)
    return out.reshape(B, T, C), aux[0, 0]


# speedup vs baseline: 1.2623x; 1.1736x over previous
import functools

import jax
import jax.numpy as jnp
from jax import lax
from jax.experimental import pallas as pl
from jax.experimental.pallas import tpu as pltpu
from jax.experimental.pallas import tpu_sc as plsc

_N = 4096
_C = 1024
_E = 8
_HID = 4096
_CAP = 640
_TBLK = 512
_NBLK = _N // _TBLK
_KBLK = 2048
_KN = _HID // _KBLK
_NC, _NS = 2, 16
_NW = _NC * _NS
_TOK_W = _N // _NW
_ROWS = _E * _CAP + _NW


def _router_body(x_ref, wr_ref, br_ref, dst_ref, g_ref, scale_ref, aux_ref,
                 counts_ref, imp_ref):
    i = pl.program_id(0)

    @pl.when(i == 0)
    def _():
        counts_ref[...] = jnp.zeros_like(counts_ref)
        imp_ref[...] = jnp.zeros_like(imp_ref)

    xb = x_ref[...]
    logits = jnp.dot(xb, wr_ref[...], preferred_element_type=jnp.float32)
    logits = logits + br_ref[...]
    m = jnp.max(logits, axis=-1, keepdims=True)
    ex = jnp.exp(logits - m)
    probs = ex / jnp.sum(ex, axis=-1, keepdims=True)

    is_max = (logits == m).astype(jnp.float32)
    le = (lax.broadcasted_iota(jnp.int32, (_E, _E), 0)
          <= lax.broadcasted_iota(jnp.int32, (_E, _E), 1)).astype(jnp.float32)
    nmax_incl = jnp.dot(is_max, le, preferred_element_type=jnp.float32)
    first = is_max * (nmax_incl == 1.0).astype(jnp.float32)

    eidx_col = lax.broadcasted_iota(jnp.int32, (_E, 1), 0).astype(jnp.float32)
    top_idx = jnp.dot(first, eidx_col, preferred_element_type=jnp.float32)
    top_val = jnp.sum(probs * first, axis=-1, keepdims=True)

    tril = (lax.broadcasted_iota(jnp.int32, (_TBLK, _TBLK), 0)
            >= lax.broadcasted_iota(jnp.int32, (_TBLK, _TBLK), 1)
            ).astype(jnp.float32)
    incl = jnp.dot(tril, first, preferred_element_type=jnp.float32)
    incl = incl + counts_ref[...]
    pos = jnp.sum(incl * first, axis=-1, keepdims=True) - 1.0

    counts_ref[...] += jnp.sum(first, axis=0, keepdims=True)
    imp_ref[...] += jnp.sum(probs, axis=0, keepdims=True)

    keep = pos < float(_CAP)
    posi = pos.astype(jnp.int32)
    eidx = top_idx.astype(jnp.int32)
    tok = i * _TBLK + lax.broadcasted_iota(jnp.int32, (_TBLK, 1), 0)
    garbage = _E * _CAP + tok // _TOK_W
    dst_ref[...] = jnp.where(keep, eidx * _CAP + posi, garbage)
    g_ref[...] = eidx * _CAP + jnp.minimum(posi, _CAP - 1)
    scale_ref[...] = jnp.where(keep, top_val, 0.0)

    @pl.when(i == _NBLK - 1)
    def _():
        load = jnp.minimum(counts_ref[...], float(_CAP)) / float(_N)
        imp = imp_ref[...] / float(_N)
        aux_ref[...] = jnp.sum(imp * load, keepdims=True) * float(_E)


def _router(xf, Wr, br2):
    return pl.pallas_call(
        _router_body,
        grid=(_NBLK,),
        in_specs=[
            pl.BlockSpec((_TBLK, _C), lambda i: (i, 0)),
            pl.BlockSpec((_C, _E), lambda i: (0, 0)),
            pl.BlockSpec((1, _E), lambda i: (0, 0)),
        ],
        out_specs=[
            pl.BlockSpec((_TBLK, 1), lambda i: (i, 0)),
            pl.BlockSpec((_TBLK, 1), lambda i: (i, 0)),
            pl.BlockSpec((_TBLK, 1), lambda i: (i, 0)),
            pl.BlockSpec((1, 1), lambda i: (0, 0)),
        ],
        out_shape=[
            jax.ShapeDtypeStruct((_N, 1), jnp.int32),
            jax.ShapeDtypeStruct((_N, 1), jnp.int32),
            jax.ShapeDtypeStruct((_N, 1), jnp.float32),
            jax.ShapeDtypeStruct((1, 1), jnp.float32),
        ],
        scratch_shapes=[
            pltpu.VMEM((1, _E), jnp.float32),
            pltpu.VMEM((1, _E), jnp.float32),
        ],
        compiler_params=pltpu.CompilerParams(
            dimension_semantics=("arbitrary",)),
    )(xf, Wr, br2)


_CHUNK = 64


@functools.lru_cache(maxsize=None)
def _build_sc_kernels():
    mesh = plsc.VectorSubcoreMesh(core_axis_name="c", subcore_axis_name="s")
    nrow = _TOK_W // _CHUNK
    scratch = [
        pltpu.VMEM((nrow, _CHUNK), jnp.int32),
        pltpu.VMEM((_CHUNK, _C), jnp.float32),
        pltpu.SemaphoreType.DMA,
    ]

    @functools.partial(
        pl.kernel, mesh=mesh,
        out_type=jax.ShapeDtypeStruct((_ROWS, _C), jnp.float32),
        scratch_types=scratch,
    )
    def sc_scatter(x_hbm, dst_hbm, buf_hbm, idx_v, rows_v, sem):
        wid = lax.axis_index("s") * _NC + lax.axis_index("c")
        pltpu.sync_copy(dst_hbm.at[pl.ds(wid * nrow, nrow)], idx_v)
        for j in range(nrow):
            base = wid * _TOK_W + j * _CHUNK
            pltpu.sync_copy(x_hbm.at[pl.ds(base, _CHUNK)], rows_v)
            pltpu.async_copy(rows_v, buf_hbm.at[idx_v.at[j]], sem).wait()

    @functools.partial(
        pl.kernel, mesh=mesh,
        out_type=jax.ShapeDtypeStruct((_N, _C), jnp.float32),
        scratch_types=scratch,
    )
    def sc_gather(rows_hbm, g_hbm, out_hbm, idx_v, rows_v, sem):
        wid = lax.axis_index("s") * _NC + lax.axis_index("c")
        pltpu.sync_copy(g_hbm.at[pl.ds(wid * nrow, nrow)], idx_v)
        for j in range(nrow):
            base = wid * _TOK_W + j * _CHUNK
            pltpu.async_copy(rows_hbm.at[idx_v.at[j]], rows_v, sem).wait()
            pltpu.sync_copy(rows_v, out_hbm.at[pl.ds(base, _CHUNK)])

    return sc_scatter, sc_gather


def _ffn_body(buf_ref, w1_ref, b1_ref, w2_ref, b2_ref, out_ref):
    k = pl.program_id(1)
    h = jnp.dot(buf_ref[...].astype(jnp.bfloat16),
                w1_ref[0].astype(jnp.bfloat16),
                preferred_element_type=jnp.float32)
    h = jnp.maximum(h + b1_ref[0], 0.0)
    contrib = jnp.dot(h.astype(jnp.bfloat16), w2_ref[0].astype(jnp.bfloat16),
                      preferred_element_type=jnp.float32)

    @pl.when(k == 0)
    def _():
        out_ref[...] = contrib + b2_ref[0]

    @pl.when(k > 0)
    def _():
        out_ref[...] += contrib


def _ffn(buf, W1, b1, W2, b2):
    return pl.pallas_call(
        _ffn_body,
        grid=(_E, _KN),
        in_specs=[
            pl.BlockSpec((_CAP, _C), lambda e, k: (e, 0)),
            pl.BlockSpec((1, _C, _KBLK), lambda e, k: (e, 0, k)),
            pl.BlockSpec((1, 1, _KBLK), lambda e, k: (e, 0, k)),
            pl.BlockSpec((1, _KBLK, _C), lambda e, k: (e, k, 0)),
            pl.BlockSpec((1, 1, _C), lambda e, k: (e, 0, 0)),
        ],
        out_specs=pl.BlockSpec((_CAP, _C), lambda e, k: (e, 0)),
        out_shape=jax.ShapeDtypeStruct((_E * _CAP, _C), jnp.float32),
        compiler_params=pltpu.CompilerParams(
            dimension_semantics=("parallel", "arbitrary")),
    )(buf, W1, b1.reshape(_E, 1, _HID), W2, b2.reshape(_E, 1, _C))


def _scale_body(x_ref, s_ref, o_ref):
    o_ref[...] = x_ref[...].astype(jnp.float32) * s_ref[...]


def _scale_mul(gathered, scale):
    return pl.pallas_call(
        _scale_body,
        grid=(_NBLK,),
        in_specs=[
            pl.BlockSpec((_TBLK, _C), lambda i: (i, 0)),
            pl.BlockSpec((_TBLK, 1), lambda i: (i, 0)),
        ],
        out_specs=pl.BlockSpec((_TBLK, _C), lambda i: (i, 0)),
        out_shape=jax.ShapeDtypeStruct((_N, _C), jnp.float32),
        compiler_params=pltpu.CompilerParams(
            dimension_semantics=("parallel",)),
    )(gathered, scale)


def kernel(x, Wr, br, W1, b1, W2, b2):
    B, T, C = x.shape
    xf = x.reshape(_N, C)
    dst, g, scale, aux = _router(xf, Wr, br.reshape(1, _E))
    dst_w = dst.reshape(_NW * (_TOK_W // _CHUNK), _CHUNK)
    g_w = g.reshape(_NW * (_TOK_W // _CHUNK), _CHUNK)
    sc_scatter, sc_gather = _build_sc_kernels()
    buf = sc_scatter(xf, dst_w)
    out_rows = _ffn(buf, W1, b1, W2, b2)
    gathered = sc_gather(out_rows, g_w)
    out = _scale_mul(gathered, scale)
    return out.reshape(B, T, C), aux[0, 0]


# speedup vs baseline: 1.3448x; 1.0654x over previous
import functools

import jax
import jax.numpy as jnp
from jax import lax
from jax.experimental import pallas as pl
from jax.experimental.pallas import tpu as pltpu
from jax.experimental.pallas import tpu_sc as plsc

_N = 4096
_C = 1024
_E = 8
_HID = 4096
_CAP = 640
_TBLK = 512
_NBLK = _N // _TBLK
_KBLK = 2048
_KN = _HID // _KBLK
_SREP = 128
_NC, _NS = 2, 16
_NW = _NC * _NS
_TOK_W = _N // _NW
_ROWS = _E * _CAP + _NW
_OCAP = _CAP + 32


def _router_body(x_ref, wr_ref, br_ref, dst_ref, g_ref, scale_ref, aux_ref,
                 counts_ref, imp_ref):
    i = pl.program_id(0)

    @pl.when(i == 0)
    def _():
        counts_ref[...] = jnp.zeros_like(counts_ref)
        imp_ref[...] = jnp.zeros_like(imp_ref)

    xb = x_ref[...]
    logits = jnp.dot(xb, wr_ref[...], preferred_element_type=jnp.float32)
    logits = logits + br_ref[...]
    m = jnp.max(logits, axis=-1, keepdims=True)
    ex = jnp.exp(logits - m)
    probs = ex / jnp.sum(ex, axis=-1, keepdims=True)

    is_max = (logits == m).astype(jnp.float32)
    le = (lax.broadcasted_iota(jnp.int32, (_E, _E), 0)
          <= lax.broadcasted_iota(jnp.int32, (_E, _E), 1)).astype(jnp.float32)
    nmax_incl = jnp.dot(is_max, le, preferred_element_type=jnp.float32)
    first = is_max * (nmax_incl == 1.0).astype(jnp.float32)

    eidx_col = lax.broadcasted_iota(jnp.int32, (_E, 1), 0).astype(jnp.float32)
    top_idx = jnp.dot(first, eidx_col, preferred_element_type=jnp.float32)
    top_val = jnp.sum(probs * first, axis=-1, keepdims=True)

    tril = (lax.broadcasted_iota(jnp.int32, (_TBLK, _TBLK), 0)
            >= lax.broadcasted_iota(jnp.int32, (_TBLK, _TBLK), 1)
            ).astype(jnp.float32)
    incl = jnp.dot(tril, first, preferred_element_type=jnp.float32)
    incl = incl + counts_ref[...]
    pos = jnp.sum(incl * first, axis=-1, keepdims=True) - 1.0

    counts_ref[...] += jnp.sum(first, axis=0, keepdims=True)
    imp_ref[...] += jnp.sum(probs, axis=0, keepdims=True)

    keep = pos < float(_CAP)
    posi = pos.astype(jnp.int32)
    eidx = top_idx.astype(jnp.int32)
    tok = i * _TBLK + lax.broadcasted_iota(jnp.int32, (_TBLK, 1), 0)
    garbage = _E * _CAP + tok // _TOK_W
    dst_ref[...] = jnp.where(keep, eidx * _CAP + posi, garbage).reshape(
        _TBLK // _CHUNK, _CHUNK)
    g_ref[...] = jnp.where(
        keep, eidx * _OCAP + posi, eidx * _OCAP + _CAP).reshape(
        _TBLK // _CHUNK, _CHUNK)
    scale = jnp.where(keep, top_val, 0.0)
    scale_ref[...] = jnp.broadcast_to(scale, (_TBLK, _SREP))

    @pl.when(i == _NBLK - 1)
    def _():
        load = jnp.minimum(counts_ref[...], float(_CAP)) / float(_N)
        imp = imp_ref[...] / float(_N)
        aux_ref[...] = jnp.sum(imp * load, keepdims=True) * float(_E)


def _router(xf, Wr, br2):
    return pl.pallas_call(
        _router_body,
        grid=(_NBLK,),
        in_specs=[
            pl.BlockSpec((_TBLK, _C), lambda i: (i, 0)),
            pl.BlockSpec((_C, _E), lambda i: (0, 0)),
            pl.BlockSpec((1, _E), lambda i: (0, 0)),
        ],
        out_specs=[
            pl.BlockSpec((_TBLK // _CHUNK, _CHUNK), lambda i: (i, 0)),
            pl.BlockSpec((_TBLK // _CHUNK, _CHUNK), lambda i: (i, 0)),
            pl.BlockSpec((_TBLK, _SREP), lambda i: (i, 0)),
            pl.BlockSpec((1, 1), lambda i: (0, 0)),
        ],
        out_shape=[
            jax.ShapeDtypeStruct((_N // _CHUNK, _CHUNK), jnp.int32),
            jax.ShapeDtypeStruct((_N // _CHUNK, _CHUNK), jnp.int32),
            jax.ShapeDtypeStruct((_N, _SREP), jnp.float32),
            jax.ShapeDtypeStruct((1, 1), jnp.float32),
        ],
        scratch_shapes=[
            pltpu.VMEM((1, _E), jnp.float32),
            pltpu.VMEM((1, _E), jnp.float32),
        ],
        compiler_params=pltpu.CompilerParams(
            dimension_semantics=("arbitrary",)),
    )(xf, Wr, br2)


_CHUNK = 32
_NCHUNK = _TOK_W // _CHUNK


@functools.lru_cache(maxsize=None)
def _build_sc_kernels():
    mesh = plsc.VectorSubcoreMesh(core_axis_name="c", subcore_axis_name="s")
    scratch = [
        pltpu.VMEM((_NCHUNK, _CHUNK), jnp.int32),
        pltpu.VMEM((_TOK_W, _SREP), jnp.float32),
        pltpu.VMEM((_TOK_W, 16), jnp.float32),
        pltpu.VMEM((_CHUNK, _C), jnp.float32),
        pltpu.VMEM((_CHUNK, _C), jnp.float32),
        pltpu.SemaphoreType.DMA,
        pltpu.SemaphoreType.DMA,
        pltpu.SemaphoreType.DMA,
        pltpu.SemaphoreType.DMA,
        pltpu.SemaphoreType.DMA,
    ]

    @functools.partial(
        pl.kernel, mesh=mesh,
        out_type=[
            jax.ShapeDtypeStruct((_ROWS, _C), jnp.float32),
            jax.ShapeDtypeStruct((_ROWS, _SREP), jnp.float32),
        ],
        scratch_types=scratch,
    )
    def sc_scatter(x_hbm, dst_hbm, srep_hbm, buf_hbm, sbuf_hbm,
                   idx_v, srep_v, snar_v, b0, b1, si0, si1, so0, so1, ss):
        wid = lax.axis_index("s") * _NC + lax.axis_index("c")
        bufs, sin, sout = [b0, b1], [si0, si1], [so0, so1]
        base = wid * _TOK_W
        pltpu.sync_copy(dst_hbm.at[pl.ds(wid * _NCHUNK, _NCHUNK)], idx_v)
        pltpu.sync_copy(srep_hbm.at[pl.ds(base, _TOK_W)], srep_v)
        loads = [None] * _NCHUNK
        scats = [None] * _NCHUNK
        sscat = [None] * _NCHUNK
        loads[0] = pltpu.async_copy(x_hbm.at[pl.ds(base, _CHUNK)], b0, si0)
        for j in range(_NCHUNK):
            if j + 1 < _NCHUNK:
                if j >= 1:
                    scats[j - 1].wait()
                loads[j + 1] = pltpu.async_copy(
                    x_hbm.at[pl.ds(base + (j + 1) * _CHUNK, _CHUNK)],
                    bufs[(j + 1) % 2], sin[(j + 1) % 2])
            loads[j].wait()
            scats[j] = pltpu.async_copy(bufs[j % 2], buf_hbm.at[idx_v.at[j]],
                                        sout[j % 2])
            sscat[j] = pltpu.async_copy(
                srep_v.at[pl.ds(j * _CHUNK, _CHUNK)],
                sbuf_hbm.at[idx_v.at[j]], ss)
        scats[_NCHUNK - 2].wait()
        scats[_NCHUNK - 1].wait()
        for j in range(_NCHUNK):
            sscat[j].wait()

    @functools.partial(
        pl.kernel, mesh=mesh,
        out_type=jax.ShapeDtypeStruct((_N, _C), jnp.float32),
        scratch_types=scratch,
    )
    def sc_gather(rows_hbm, g_hbm, out_hbm,
                  idx_v, srep_v, snar_v, b0, b1, si0, si1, so0, so1, ss):
        wid = lax.axis_index("s") * _NC + lax.axis_index("c")
        bufs, sin, sout = [b0, b1], [si0, si1], [so0, so1]
        base = wid * _TOK_W
        pltpu.sync_copy(g_hbm.at[pl.ds(wid * _NCHUNK, _NCHUNK)], idx_v)
        glds = [None] * _NCHUNK
        sts = [None] * _NCHUNK
        glds[0] = pltpu.async_copy(rows_hbm.at[idx_v.at[0]], b0, si0)
        for j in range(_NCHUNK):
            if j + 1 < _NCHUNK:
                if j >= 1:
                    sts[j - 1].wait()
                glds[j + 1] = pltpu.async_copy(rows_hbm.at[idx_v.at[j + 1]],
                                               bufs[(j + 1) % 2],
                                               sin[(j + 1) % 2])
            glds[j].wait()
            sts[j] = pltpu.async_copy(
                bufs[j % 2], out_hbm.at[pl.ds(base + j * _CHUNK, _CHUNK)],
                sout[j % 2])
        sts[_NCHUNK - 2].wait()
        sts[_NCHUNK - 1].wait()

    return sc_scatter, sc_gather


def _ffn_body(buf_ref, w1_ref, b1_ref, w2_ref, b2_ref, s_ref, out_ref):
    k = pl.program_id(1)
    h = jnp.dot(buf_ref[...], w1_ref[0],
                preferred_element_type=jnp.float32)
    h = jnp.maximum(h + b1_ref[0], 0.0)
    contrib = jnp.dot(h, w2_ref[0],
                      preferred_element_type=jnp.float32)

    @pl.when(k == 0)
    def _():
        out_ref[pl.ds(0, _CAP), :] = contrib + b2_ref[0]
        out_ref[pl.ds(_CAP, _OCAP - _CAP), :] = jnp.zeros(
            (_OCAP - _CAP, _C), jnp.float32)

    @pl.when((k > 0) & (k < _KN - 1))
    def _():
        out_ref[pl.ds(0, _CAP), :] += contrib

    @pl.when(k == _KN - 1)
    def _():
        out_ref[pl.ds(0, _CAP), :] = (
            out_ref[pl.ds(0, _CAP), :] + contrib) * s_ref[:, :1]


def _ffn(buf, W1, b1, W2, b2, sbuf):
    return pl.pallas_call(
        _ffn_body,
        grid=(_E, _KN),
        in_specs=[
            pl.BlockSpec((_CAP, _C), lambda e, k: (e, 0)),
            pl.BlockSpec((1, _C, _KBLK), lambda e, k: (e, 0, k)),
            pl.BlockSpec((1, 1, _KBLK), lambda e, k: (e, 0, k)),
            pl.BlockSpec((1, _KBLK, _C), lambda e, k: (e, k, 0)),
            pl.BlockSpec((1, 1, _C), lambda e, k: (e, 0, 0)),
            pl.BlockSpec((_CAP, _SREP), lambda e, k: (e, 0)),
        ],
        out_specs=pl.BlockSpec((_OCAP, _C), lambda e, k: (e, 0)),
        out_shape=jax.ShapeDtypeStruct((_E * _OCAP, _C), jnp.float32),
        compiler_params=pltpu.CompilerParams(
            dimension_semantics=("parallel", "arbitrary")),
    )(buf, W1, b1.reshape(_E, 1, _HID), W2, b2.reshape(_E, 1, _C), sbuf)


def kernel(x, Wr, br, W1, b1, W2, b2):
    B, T, C = x.shape
    xf = x.reshape(_N, C)
    dst, g, srep, aux = _router(xf, Wr, br.reshape(1, _E))
    sc_scatter, sc_gather = _build_sc_kernels()
    buf, sbuf = sc_scatter(xf, dst, srep)
    out_rows = _ffn(buf, W1, b1, W2, b2, sbuf)
    out = sc_gather(out_rows, g)
    return out.reshape(B, T, C), aux[0, 0]
